# Initial kernel scaffold; baseline (speedup 1.0000x reference)
import jax
import jax.numpy as jnp
from jax.experimental import pallas as pl


def kernel(points, point_indices, memory_bank):
    raise NotImplementedError("write your pallas kernel here")


# speedup vs baseline: 36.9471x; 36.9471x over previous
import jax
import jax.numpy as jnp
from jax.experimental import pallas as pl
from jax.experimental.pallas import tpu as pltpu

_K = 4096
_B = 1024
_D = 32
_M = 100000
_RBS = 64
_NRB = _B // _RBS
_MT = 2048
_NMT = (_M + _MT - 1) // _MT
_MPAD = _NMT * _MT
_CHUNK = 512
_NCHUNK = _MPAD // _CHUNK
_CPM = _MT // _CHUNK
_NEG = -1e30
_BNEG = -3.0e38
_PASSES = 10


def _fold(x, op):
    return op(x.reshape(_RBS, _MT // 128, 128), axis=1)


def _body(pts_ref, idx_ref, bank_ref, out_ref, loss_ref,
          npts_s, bf_s, posv_s, rmaxv_s, rminv_s, iota_s, idxb_s):
    rb = pl.program_id(0)
    mt = pl.program_id(1)

    @pl.when(jnp.logical_and(rb == 0, mt == 0))
    def _init_iota():
        iota_s[...] = jax.lax.broadcasted_iota(
            jnp.int32, (_RBS, _MT), 1).astype(jnp.float32)

    @pl.when(mt == 0)
    def _init():
        p = pts_ref[...]
        n = p / jnp.sqrt(jnp.sum(p * p, axis=1, keepdims=True))
        npts_s[...] = n.astype(jnp.bfloat16)
        posv_s[...] = jnp.zeros_like(posv_s)
        rmaxv_s[...] = jnp.full_like(rmaxv_s, _NEG)
        rminv_s[...] = jnp.full_like(rminv_s, -_NEG)
        idxb_s[...] = idx_ref[...].astype(jnp.float32) + jnp.zeros(
            (_RBS, _MT), jnp.float32)

    sims = jax.lax.dot_general(
        npts_s[...], bank_ref[...], (((1,), (0,)), ((), ())),
        preferred_element_type=jnp.float32)
    out_ref[...] = sims

    iota = iota_s[...]
    pmask = iota == (idxb_s[...] - jnp.float32(mt * _MT))
    posv_s[...] += _fold(jnp.where(pmask, sims, 0.0), jnp.sum)

    @pl.when(mt < _NMT - 1)
    def _mid_tile():
        rmaxv_s[...] = jnp.maximum(rmaxv_s[...], _fold(sims, jnp.max))
        rminv_s[...] = jnp.minimum(rminv_s[...], _fold(sims, jnp.min))
        sb = sims.astype(jnp.bfloat16)
        for j in range(_CPM):
            bf_s[mt * _CPM + j] = sb[:, j * _CHUNK:(j + 1) * _CHUNK]

    @pl.when(mt == _NMT - 1)
    def _last_tile():
        valid = iota < jnp.float32(_M - (_NMT - 1) * _MT)
        smax = jnp.where(valid, sims, _NEG)
        smin = jnp.where(valid, sims, -_NEG)
        rmaxv_s[...] = jnp.maximum(rmaxv_s[...], _fold(smax, jnp.max))
        rminv_s[...] = jnp.minimum(rminv_s[...], _fold(smin, jnp.min))
        sb = jnp.where(valid, sims, _BNEG).astype(jnp.bfloat16)
        for j in range(_CPM):
            bf_s[(_NMT - 1) * _CPM + j] = sb[:, j * _CHUNK:(j + 1) * _CHUNK]

        rmax = jnp.max(rmaxv_s[...], axis=1, keepdims=True)
        rmin = jnp.min(rminv_s[...], axis=1, keepdims=True)
        pos = jnp.sum(posv_s[...], axis=1, keepdims=True)

        one_b = jnp.ones((), jnp.bfloat16)
        zero_b = jnp.zeros((), jnp.bfloat16)

        def pass_body(_, carry):
            lo, hi = carry
            mid = 0.5 * (lo + hi)
            midb = (mid + jnp.zeros((_RBS, _CHUNK), jnp.float32)).astype(
                jnp.bfloat16)

            def chunk_body(c, acc):
                v = bf_s[c]
                return acc + jnp.where(v > midb, one_b, zero_b)

            acc = jax.lax.fori_loop(
                0, _NCHUNK, chunk_body,
                jnp.zeros((_RBS, _CHUNK), jnp.bfloat16))
            cnt = jnp.sum(acc.astype(jnp.float32), axis=1, keepdims=True)
            ge = cnt >= _K
            return jnp.where(ge, mid, lo), jnp.where(ge, hi, mid)

        lo, hi = jax.lax.fori_loop(0, _PASSES, pass_body, (rmin, rmax))

        lob = lo + jnp.zeros((_RBS, _CHUNK // 2), jnp.float32)

        def sum_body(c, slo):
            v = bf_s[c]
            for j in range(2):
                vh = v[:, j * (_CHUNK // 2):(j + 1) * (_CHUNK // 2)]
                slo = slo + jnp.maximum(vh.astype(jnp.float32) - lob, 0.0)
            return slo

        slo = jax.lax.fori_loop(
            0, _NCHUNK, sum_body,
            jnp.zeros((_RBS, _CHUNK // 2), jnp.float32))
        sum_topk = _K * lo + jnp.sum(slo, axis=1, keepdims=True)
        loss_ref[...] = -pos + sum_topk * (1.0 / _K)


def kernel(points, point_indices, memory_bank):
    idx2 = point_indices.reshape(_B, 1)
    bank_t = memory_bank.T.astype(jnp.bfloat16)
    sims, loss_terms = pl.pallas_call(
        _body,
        grid=(_NRB, _NMT),
        in_specs=[
            pl.BlockSpec((_RBS, _D), lambda rb, mt: (rb, 0)),
            pl.BlockSpec((_RBS, 1), lambda rb, mt: (rb, 0)),
            pl.BlockSpec((_D, _MT), lambda rb, mt: (0, mt)),
        ],
        out_specs=[
            pl.BlockSpec((_RBS, _MT), lambda rb, mt: (rb, mt)),
            pl.BlockSpec((_RBS, 1), lambda rb, mt: (rb, 0)),
        ],
        out_shape=[
            jax.ShapeDtypeStruct((_B, _M), jnp.float32),
            jax.ShapeDtypeStruct((_B, 1), jnp.float32),
        ],
        scratch_shapes=[
            pltpu.VMEM((_RBS, _D), jnp.bfloat16),
            pltpu.VMEM((_NCHUNK, _RBS, _CHUNK), jnp.bfloat16),
            pltpu.VMEM((_RBS, 128), jnp.float32),
            pltpu.VMEM((_RBS, 128), jnp.float32),
            pltpu.VMEM((_RBS, 128), jnp.float32),
            pltpu.VMEM((_RBS, _MT), jnp.float32),
            pltpu.VMEM((_RBS, _MT), jnp.float32),
        ],
        compiler_params=pltpu.CompilerParams(
            dimension_semantics=("parallel", "arbitrary"),
        ),
    )(points, idx2, bank_t)
    loss = jnp.mean(loss_terms)
    return (loss, sims)


# speedup vs baseline: 39.6292x; 1.0726x over previous
import jax
import jax.numpy as jnp
from jax.experimental import pallas as pl
from jax.experimental.pallas import tpu as pltpu

_K = 4096
_B = 1024
_D = 32
_M = 100000
_RBS = 64
_NRB = _B // _RBS
_MT = 2048
_NMT = (_M + _MT - 1) // _MT
_MPAD = _NMT * _MT
_CHUNK = 512
_NCHUNK = _MPAD // _CHUNK
_CPM = _MT // _CHUNK
_NEG = -1e30
_BNEG = -3.0e38
_PASSES = 10


def _fold(x, op2):
    parts = [x[:, j * 128:(j + 1) * 128] for j in range(x.shape[1] // 128)]
    while len(parts) > 1:
        parts = [op2(parts[i], parts[i + 1]) if i + 1 < len(parts)
                 else parts[i] for i in range(0, len(parts), 2)]
    return parts[0]


def _body(pts_ref, idx_ref, bank_ref, out_ref, loss_ref,
          npts_s, bf_s, posv_s, rmaxv_s, rminv_s, iota_s, idxb_s):
    rb = pl.program_id(0)
    mt = pl.program_id(1)

    @pl.when(jnp.logical_and(rb == 0, mt == 0))
    def _init_iota():
        iota_s[...] = jax.lax.broadcasted_iota(
            jnp.int32, (_RBS, _MT), 1).astype(jnp.float32)

    @pl.when(mt == 0)
    def _init():
        p = pts_ref[...]
        n = p / jnp.sqrt(jnp.sum(p * p, axis=1, keepdims=True))
        npts_s[...] = n.astype(jnp.bfloat16)
        posv_s[...] = jnp.zeros_like(posv_s)
        rmaxv_s[...] = jnp.full_like(rmaxv_s, _NEG)
        rminv_s[...] = jnp.full_like(rminv_s, -_NEG)
        idxb_s[...] = idx_ref[...].astype(jnp.float32) + jnp.zeros(
            (_RBS, _MT), jnp.float32)

    sims = jax.lax.dot_general(
        npts_s[...], bank_ref[...], (((1,), (0,)), ((), ())),
        preferred_element_type=jnp.float32)
    out_ref[...] = sims

    iota = iota_s[...]
    pmask = iota == (idxb_s[...] - jnp.float32(mt * _MT))
    posv_s[...] += _fold(jnp.where(pmask, sims, 0.0), jnp.add)

    @pl.when(mt < _NMT - 1)
    def _mid_tile():
        rmaxv_s[...] = jnp.maximum(rmaxv_s[...], _fold(sims, jnp.maximum))
        rminv_s[...] = jnp.minimum(rminv_s[...], _fold(sims, jnp.minimum))
        sb = sims.astype(jnp.bfloat16)
        for j in range(_CPM):
            bf_s[mt * _CPM + j] = sb[:, j * _CHUNK:(j + 1) * _CHUNK]

    @pl.when(mt == _NMT - 1)
    def _last_tile():
        valid = iota < jnp.float32(_M - (_NMT - 1) * _MT)
        smax = jnp.where(valid, sims, _NEG)
        smin = jnp.where(valid, sims, -_NEG)
        rmaxv_s[...] = jnp.maximum(rmaxv_s[...], _fold(smax, jnp.maximum))
        rminv_s[...] = jnp.minimum(rminv_s[...], _fold(smin, jnp.minimum))
        sb = jnp.where(valid, sims, _BNEG).astype(jnp.bfloat16)
        for j in range(_CPM):
            bf_s[(_NMT - 1) * _CPM + j] = sb[:, j * _CHUNK:(j + 1) * _CHUNK]

        rmax = jnp.max(rmaxv_s[...], axis=1, keepdims=True)
        rmin = jnp.min(rminv_s[...], axis=1, keepdims=True)
        pos = jnp.sum(posv_s[...], axis=1, keepdims=True)

        one_b = jnp.ones((), jnp.bfloat16)
        zero_b = jnp.zeros((), jnp.bfloat16)

        def pass_body(_, carry):
            lo, hi = carry
            mid = 0.5 * (lo + hi)
            midb = (mid + jnp.zeros((_RBS, _CHUNK), jnp.float32)).astype(
                jnp.bfloat16)

            def chunk_body(c, acc):
                v = bf_s[c]
                return acc + jnp.where(v > midb, one_b, zero_b)

            acc = jax.lax.fori_loop(
                0, _NCHUNK, chunk_body,
                jnp.zeros((_RBS, _CHUNK), jnp.bfloat16))
            cnt = jnp.sum(acc.astype(jnp.float32), axis=1, keepdims=True)
            ge = cnt >= _K
            return jnp.where(ge, mid, lo), jnp.where(ge, hi, mid)

        lo, hi = jax.lax.fori_loop(0, _PASSES, pass_body, (rmin, rmax))

        lob = lo + jnp.zeros((_RBS, _CHUNK // 2), jnp.float32)

        def sum_body(c, slo):
            v = bf_s[c]
            for j in range(2):
                vh = v[:, j * (_CHUNK // 2):(j + 1) * (_CHUNK // 2)]
                slo = slo + jnp.maximum(vh.astype(jnp.float32) - lob, 0.0)
            return slo

        slo = jax.lax.fori_loop(
            0, _NCHUNK, sum_body,
            jnp.zeros((_RBS, _CHUNK // 2), jnp.float32))
        sum_topk = _K * lo + jnp.sum(slo, axis=1, keepdims=True)
        loss_ref[...] = -pos + sum_topk * (1.0 / _K)


def kernel(points, point_indices, memory_bank):
    idx2 = point_indices.reshape(_B, 1)
    bank_t = memory_bank.T.astype(jnp.bfloat16)
    sims, loss_terms = pl.pallas_call(
        _body,
        grid=(_NRB, _NMT),
        in_specs=[
            pl.BlockSpec((_RBS, _D), lambda rb, mt: (rb, 0)),
            pl.BlockSpec((_RBS, 1), lambda rb, mt: (rb, 0)),
            pl.BlockSpec((_D, _MT), lambda rb, mt: (0, mt)),
        ],
        out_specs=[
            pl.BlockSpec((_RBS, _MT), lambda rb, mt: (rb, mt)),
            pl.BlockSpec((_RBS, 1), lambda rb, mt: (rb, 0)),
        ],
        out_shape=[
            jax.ShapeDtypeStruct((_B, _M), jnp.float32),
            jax.ShapeDtypeStruct((_B, 1), jnp.float32),
        ],
        scratch_shapes=[
            pltpu.VMEM((_RBS, _D), jnp.bfloat16),
            pltpu.VMEM((_NCHUNK, _RBS, _CHUNK), jnp.bfloat16),
            pltpu.VMEM((_RBS, 128), jnp.float32),
            pltpu.VMEM((_RBS, 128), jnp.float32),
            pltpu.VMEM((_RBS, 128), jnp.float32),
            pltpu.VMEM((_RBS, _MT), jnp.float32),
            pltpu.VMEM((_RBS, _MT), jnp.float32),
        ],
        compiler_params=pltpu.CompilerParams(
            dimension_semantics=("parallel", "arbitrary"),
        ),
    )(points, idx2, bank_t)
    loss = jnp.mean(loss_terms)
    return (loss, sims)
